# Initial kernel scaffold; baseline (speedup 1.0000x reference)
import jax
import jax.numpy as jnp
from jax.experimental import pallas as pl


def kernel(X, A_tilde, W1, W2):
    raise NotImplementedError("write your pallas kernel here")


# speedup vs baseline: 20.0860x; 20.0860x over previous
import math

import jax
import jax.numpy as jnp
from jax import lax
from jax.experimental import pallas as pl
from jax.experimental.pallas import tpu as pltpu


def _linear_kernel(x_ref, w_ref, o_ref):
    o_ref[...] = lax.dot_general(
        x_ref[...], w_ref[...], (((1,), (1,)), ((), ())),
        preferred_element_type=jnp.float32)


def _layer1_kernel(a_ref, u_ref, w2_ref, o_ref):
    h = lax.dot_general(
        a_ref[...], u_ref[...], (((1,), (0,)), ((), ())),
        preferred_element_type=jnp.float32)
    h = jnp.maximum(h, 0.0)
    o_ref[...] = lax.dot_general(
        h, w2_ref[...], (((1,), (1,)), ((), ())),
        preferred_element_type=jnp.float32)


def _layer2_kernel(a_ref, v_ref, o_ref):
    o_ref[...] = lax.dot_general(
        a_ref[...], v_ref[...], (((1,), (0,)), ((), ())),
        preferred_element_type=jnp.float32)


def _decoder_kernel(zi_ref, z_ref, o_ref):
    logits = lax.dot_general(
        zi_ref[...], z_ref[...], (((1,), (1,)), ((), ())),
        preferred_element_type=jnp.float32)
    o_ref[...] = jax.nn.sigmoid(logits)


def kernel(X, A_tilde, W1, W2):
    N, _ = X.shape
    H = W1.shape[0]
    L = W2.shape[0]

    bm = 400 if N % 400 == 0 else math.gcd(N, 8 * 10**6)

    U = pl.pallas_call(
        _linear_kernel,
        out_shape=jax.ShapeDtypeStruct((N, H), jnp.float32),
    )(X, W1)

    row_params = pltpu.CompilerParams(dimension_semantics=("parallel",))
    grid = (N // bm,)

    V = pl.pallas_call(
        _layer1_kernel,
        grid=grid,
        in_specs=[
            pl.BlockSpec((bm, N), lambda i: (i, 0)),
            pl.BlockSpec((N, H), lambda i: (0, 0)),
            pl.BlockSpec((L, H), lambda i: (0, 0)),
        ],
        out_specs=pl.BlockSpec((bm, L), lambda i: (i, 0)),
        out_shape=jax.ShapeDtypeStruct((N, L), jnp.float32),
        compiler_params=row_params,
    )(A_tilde, U, W2)

    Z = pl.pallas_call(
        _layer2_kernel,
        grid=grid,
        in_specs=[
            pl.BlockSpec((bm, N), lambda i: (i, 0)),
            pl.BlockSpec((N, L), lambda i: (0, 0)),
        ],
        out_specs=pl.BlockSpec((bm, L), lambda i: (i, 0)),
        out_shape=jax.ShapeDtypeStruct((N, L), jnp.float32),
        compiler_params=row_params,
    )(A_tilde, V)

    A_hat = pl.pallas_call(
        _decoder_kernel,
        grid=grid,
        in_specs=[
            pl.BlockSpec((bm, L), lambda i: (i, 0)),
            pl.BlockSpec((N, L), lambda i: (0, 0)),
        ],
        out_specs=pl.BlockSpec((bm, N), lambda i: (i, 0)),
        out_shape=jax.ShapeDtypeStruct((N, N), jnp.float32),
        compiler_params=row_params,
    )(Z, Z)

    return (A_hat, jnp.array([0]), jnp.array([0]))
